# baseline (speedup 1.0000x reference)
import functools

import jax
import jax.numpy as jnp
from jax import lax
from jax.experimental import pallas as pl
from jax.experimental.pallas import tpu as pltpu
from jax.experimental.pallas import tpu_sc as plsc

CAP = 10000
PAD = 10240
NW = 16
SLOTS_W = PAD // NW
NV = SLOTS_W // 16
B = 1024
L = 16
K = 32
SEQ = 32
HID = 64
NEG = -3.0e38
BIGI = 1 << 30

_mesh = plsc.VectorSubcoreMesh(
    core_axis_name="c", subcore_axis_name="s", num_cores=2, num_subcores=16
)


@functools.partial(
    pl.kernel,
    out_type=jax.ShapeDtypeStruct((K, SEQ, HID), jnp.float32),
    mesh=_mesh,
    scratch_types=[
        pltpu.VMEM((B,), jnp.int32),
        pltpu.VMEM((B,), jnp.float32),
        pltpu.VMEM((B,), jnp.float32),
        pltpu.VMEM((SLOTS_W,), jnp.float32),
        pltpu.VMEM((SLOTS_W,), jnp.int32),
        pltpu.VMEM((2 * L,), jnp.int32),
        pltpu.VMEM((3 * L,), jnp.float32),
        pltpu.VMEM((NW * K,), jnp.float32),
        pltpu.VMEM((NW * K,), jnp.int32),
        pltpu.VMEM((PAD,), jnp.int32),
        pltpu.VMEM((4 * L,), jnp.int32),
        pltpu.VMEM((SEQ, HID), jnp.float32),
        pltpu.VMEM_SHARED((NW * K,), jnp.float32),
        pltpu.VMEM_SHARED((NW * K,), jnp.int32),
        pltpu.VMEM_SHARED((PAD,), jnp.int32),
        pltpu.VMEM_SHARED((4 * L,), jnp.int32),
        pltpu.SemaphoreType.DMA,
    ],
    compiler_params=pltpu.CompilerParams(needs_layout_passes=False),
)
def _sc_replay(idx_hbm, loss_hbm, sur_hbm, pri_hbm, hs_hbm, mem_hbm, out_hbm,
               idx_v, loss_v, sur_v, np_v, win_v, shift_v, l1_v,
               cvals_v, cslots_v, winall_v, res_v, rowbuf_v,
               sh_cvals, sh_cslots, sh_win, sh_res, sem_row):
    cid = lax.axis_index("c")
    wid = lax.axis_index("s")
    lo = wid * SLOTS_W
    lane = lax.iota(jnp.int32, L)

    @pl.when(cid == 0)
    def _phase1():
        pltpu.sync_copy(idx_hbm, idx_v)
        pltpu.sync_copy(loss_hbm, loss_v)
        pltpu.sync_copy(sur_hbm, sur_v)
        pltpu.sync_copy(pri_hbm.at[pl.ds(lo, SLOTS_W)], np_v)

        neg1 = jnp.full((L,), -1, jnp.int32)
        for v in range(NV):
            win_v[pl.ds(v * L, L)] = neg1

        shift_v[pl.ds(L, L)] = neg1
        for c in range(B // L):
            iv = idx_v[pl.ds(c * L, L)]
            pv = (1.0 + loss_v[pl.ds(c * L, L)]) + sur_v[pl.ds(c * L, L)]
            key = iv * L + lane
            sk, spv = plsc.sort_key_val(key, pv)
            sidx = sk >> 4
            sj = (sk & (L - 1)) + (c * L)
            shift_v[pl.ds(0, L)] = sidx
            nxt = shift_v[pl.ds(1, L)]
            keep = sidx != nxt
            rel = sidx - lo
            m = keep & (rel >= 0) & (rel < SLOTS_W)
            relc = jnp.clip(rel, 0, SLOTS_W - 1)
            plsc.store_scatter(np_v, [relc], spv, mask=m)
            plsc.store_scatter(win_v, [relc], sj, mask=m)

        pltpu.sync_copy(win_v, sh_win.at[pl.ds(lo, SLOTS_W)])

        l1c = [jnp.full((L,), NEG) for _ in range(3)]
        for v in range(NV):
            mv = jnp.max(np_v[pl.ds(v * L, L)])
            l1c[v // L] = jnp.where(lane == (v % L), mv, l1c[v // L])
        for t in range(3):
            l1_v[pl.ds(t * L, L)] = l1c[t]

        lvc = [jnp.full((L,), NEG), jnp.full((L,), NEG)]
        lsc = [jnp.full((L,), -1, jnp.int32), jnp.full((L,), -1, jnp.int32)]
        for k in range(K):
            a0 = l1_v[pl.ds(0, L)]
            a1 = l1_v[pl.ds(L, L)]
            a2 = l1_v[pl.ds(2 * L, L)]
            m = jnp.max(jnp.maximum(jnp.maximum(a0, a1), a2))
            vsel = jnp.min(jnp.minimum(
                jnp.minimum(
                    jnp.where(a0 == m, lane, BIGI),
                    jnp.where(a1 == m, lane + L, BIGI)),
                jnp.where(a2 == m, lane + 2 * L, BIGI)))
            voff = vsel * L
            vv = np_v[pl.ds(voff, L)]
            lsel = jnp.min(jnp.where(vv == m, lane, BIGI))
            slot = lo + voff + lsel
            lvc[k // L] = jnp.where(lane == (k % L), m, lvc[k // L])
            lsc[k // L] = jnp.where(lane == (k % L), slot, lsc[k // L])
            vv = jnp.where(lane == lsel, NEG, vv)
            np_v[pl.ds(voff, L)] = vv
            newmax = jnp.max(vv)
            toff = (vsel // L) * L
            lchunk = l1_v[pl.ds(toff, L)]
            l1_v[pl.ds(toff, L)] = jnp.where(
                lane == (vsel - toff), newmax, lchunk)

        cvals_v[pl.ds(0, L)] = lvc[0]
        cvals_v[pl.ds(L, L)] = lvc[1]
        cslots_v[pl.ds(0, L)] = lsc[0]
        cslots_v[pl.ds(L, L)] = lsc[1]
        pltpu.sync_copy(cvals_v.at[pl.ds(0, K)], sh_cvals.at[pl.ds(wid * K, K)])
        pltpu.sync_copy(cslots_v.at[pl.ds(0, K)], sh_cslots.at[pl.ds(wid * K, K)])

    plsc.subcore_barrier()

    @pl.when((cid == 0) & (wid == 0))
    def _phase5():
        pltpu.sync_copy(sh_cvals, cvals_v)
        pltpu.sync_copy(sh_cslots, cslots_v)
        pltpu.sync_copy(sh_win, winall_v)
        ptr = jnp.zeros((L,), jnp.int32)
        rsc = [jnp.full((L,), -1, jnp.int32), jnp.full((L,), -1, jnp.int32)]
        for k in range(K):
            hidx = jnp.minimum(lane * K + ptr, NW * K - 1)
            live = ptr < K
            hv = jnp.where(live, plsc.load_gather(cvals_v, [hidx]), NEG)
            m = jnp.max(hv)
            hsel = jnp.min(jnp.where(hv == m, hidx, BIGI))
            sv = plsc.load_gather(cslots_v, [hidx])
            taken = hidx == hsel
            slot = jnp.max(jnp.where(taken, sv, -1))
            rsc[k // L] = jnp.where(lane == (k % L), slot, rsc[k // L])
            ptr = ptr + jnp.where(taken, 1, 0)
        res_v[pl.ds(0, L)] = rsc[0]
        res_v[pl.ds(L, L)] = rsc[1]
        res_v[pl.ds(2 * L, L)] = plsc.load_gather(winall_v, [rsc[0]])
        res_v[pl.ds(3 * L, L)] = plsc.load_gather(winall_v, [rsc[1]])
        pltpu.sync_copy(res_v, sh_res)

    plsc.subcore_barrier()

    @pl.when(cid == 0)
    def _phase6():
        pltpu.sync_copy(sh_res, res_v)
        for half in range(2):
            row = wid + half * L
            schunk = res_v[pl.ds(half * L, L)]
            wchunk = res_v[pl.ds((2 + half) * L, L)]
            slot = jnp.max(jnp.where(lane == wid, schunk, -1))
            win = jnp.max(jnp.where(lane == wid, wchunk, -1))

            @pl.when(win >= 0)
            def _():
                pltpu.make_async_copy(
                    hs_hbm.at[win], out_hbm.at[row], sem_row).start()

            @pl.when(win < 0)
            def _():
                pltpu.make_async_copy(
                    mem_hbm.at[slot], out_hbm.at[row], sem_row).start()

        for half in range(2):
            pltpu.make_async_copy(
                hs_hbm.at[0], out_hbm.at[0], sem_row).wait()


def kernel(hidden_states, loss, surprise, mem, priorities, idx, targets):
    del targets
    pri_pad = jnp.concatenate(
        [priorities, jnp.zeros((PAD - CAP,), jnp.float32)])
    return _sc_replay(idx.astype(jnp.int32), loss, surprise, pri_pad,
                      hidden_states, mem)


# speedup vs baseline: 1.0917x; 1.0917x over previous
import functools

import jax
import jax.numpy as jnp
from jax import lax
from jax.experimental import pallas as pl
from jax.experimental.pallas import tpu as pltpu
from jax.experimental.pallas import tpu_sc as plsc

CAP = 10000
PAD = 10240
NW = 16
SLOTS_W = PAD // NW
NV = SLOTS_W // 16
B = 1024
L = 16
K = 32
SEQ = 32
HID = 64
NEG = -3.0e38
BIGI = 1 << 30

_mesh = plsc.VectorSubcoreMesh(
    core_axis_name="c", subcore_axis_name="s", num_cores=2, num_subcores=16
)


@functools.partial(
    pl.kernel,
    out_type=jax.ShapeDtypeStruct((K, SEQ, HID), jnp.float32),
    mesh=_mesh,
    scratch_types=[
        pltpu.VMEM((B,), jnp.int32),
        pltpu.VMEM((B,), jnp.float32),
        pltpu.VMEM((B,), jnp.float32),
        pltpu.VMEM((SLOTS_W,), jnp.float32),
        pltpu.VMEM((SLOTS_W,), jnp.int32),
        pltpu.VMEM((2 * L,), jnp.int32),
        pltpu.VMEM((3 * L,), jnp.float32),
        pltpu.VMEM((NW * K,), jnp.float32),
        pltpu.VMEM((NW * K,), jnp.int32),
        pltpu.VMEM((PAD,), jnp.int32),
        pltpu.VMEM((4 * L,), jnp.int32),
        pltpu.VMEM((SEQ, HID), jnp.float32),
        pltpu.VMEM((SEQ, HID), jnp.float32),
        pltpu.VMEM_SHARED((NW * K,), jnp.float32),
        pltpu.VMEM_SHARED((NW * K,), jnp.int32),
        pltpu.VMEM_SHARED((PAD,), jnp.int32),
        pltpu.VMEM_SHARED((4 * L,), jnp.int32),
        pltpu.SemaphoreType.DMA,
        pltpu.SemaphoreType.DMA,
        pltpu.SemaphoreType.DMA,
    ],
    compiler_params=pltpu.CompilerParams(needs_layout_passes=False),
)
def _sc_replay(idx_hbm, loss_hbm, sur_hbm, pri_hbm, hs_hbm, mem_hbm, out_hbm,
               idx_v, loss_v, sur_v, np_v, win_v, shift_v, l1_v,
               cvals_v, cslots_v, winall_v, res_v, rowbuf_v, rowbuf2_v,
               sh_cvals, sh_cslots, sh_win, sh_res,
               sem_in, sem_pri, sem_row):
    cid = lax.axis_index("c")
    wid = lax.axis_index("s")
    lo = wid * SLOTS_W
    lane = lax.iota(jnp.int32, L)

    TAIL = CAP - (NW - 1) * SLOTS_W

    @pl.when(cid == 0)
    def _phase1():
        pltpu.make_async_copy(idx_hbm, idx_v, sem_in).start()
        pltpu.make_async_copy(loss_hbm, loss_v, sem_in).start()
        pltpu.make_async_copy(sur_hbm, sur_v, sem_in).start()

        @pl.when(wid == NW - 1)
        def _():
            zf = jnp.zeros((L,), jnp.float32)
            for v in range(TAIL // L, NV):
                np_v[pl.ds(v * L, L)] = zf
            pltpu.make_async_copy(
                pri_hbm.at[pl.ds(lo, TAIL)], np_v.at[pl.ds(0, TAIL)],
                sem_pri).start()

        @pl.when(wid < NW - 1)
        def _():
            pltpu.make_async_copy(
                pri_hbm.at[pl.ds(lo, SLOTS_W)], np_v, sem_pri).start()

        neg1 = jnp.full((L,), -1, jnp.int32)
        for v in range(NV):
            win_v[pl.ds(v * L, L)] = neg1
        shift_v[pl.ds(L, L)] = neg1

        pltpu.make_async_copy(idx_hbm, idx_v, sem_in).wait()
        pltpu.make_async_copy(loss_hbm, loss_v, sem_in).wait()
        pltpu.make_async_copy(sur_hbm, sur_v, sem_in).wait()

        @pl.when(wid == NW - 1)
        def _():
            pltpu.make_async_copy(
                pri_hbm.at[pl.ds(0, TAIL)], np_v.at[pl.ds(0, TAIL)],
                sem_pri).wait()

        @pl.when(wid < NW - 1)
        def _():
            pltpu.make_async_copy(
                pri_hbm.at[pl.ds(0, SLOTS_W)], np_v, sem_pri).wait()

        for c in range(B // L):
            iv = idx_v[pl.ds(c * L, L)]
            pv = (1.0 + loss_v[pl.ds(c * L, L)]) + sur_v[pl.ds(c * L, L)]
            key = iv * L + lane
            sk, spv = plsc.sort_key_val(key, pv)
            sidx = sk >> 4
            sj = (sk & (L - 1)) + (c * L)
            shift_v[pl.ds(0, L)] = sidx
            nxt = shift_v[pl.ds(1, L)]
            keep = sidx != nxt
            rel = sidx - lo
            m = keep & (rel >= 0) & (rel < SLOTS_W)
            relc = jnp.clip(rel, 0, SLOTS_W - 1)
            plsc.store_scatter(np_v, [relc], spv, mask=m)
            plsc.store_scatter(win_v, [relc], sj, mask=m)

        pltpu.sync_copy(win_v, sh_win.at[pl.ds(lo, SLOTS_W)])

        l1c = [jnp.full((L,), NEG) for _ in range(3)]
        for v in range(NV):
            mv = jnp.max(np_v[pl.ds(v * L, L)])
            l1c[v // L] = jnp.where(lane == (v % L), mv, l1c[v // L])
        for t in range(3):
            l1_v[pl.ds(t * L, L)] = l1c[t]

        lvc = [jnp.full((L,), NEG), jnp.full((L,), NEG)]
        lsc = [jnp.full((L,), -1, jnp.int32), jnp.full((L,), -1, jnp.int32)]
        for k in range(K):
            a0 = l1_v[pl.ds(0, L)]
            a1 = l1_v[pl.ds(L, L)]
            a2 = l1_v[pl.ds(2 * L, L)]
            m = jnp.max(jnp.maximum(jnp.maximum(a0, a1), a2))
            vsel = jnp.min(jnp.minimum(
                jnp.minimum(
                    jnp.where(a0 == m, lane, BIGI),
                    jnp.where(a1 == m, lane + L, BIGI)),
                jnp.where(a2 == m, lane + 2 * L, BIGI)))
            voff = vsel * L
            vv = np_v[pl.ds(voff, L)]
            lsel = jnp.min(jnp.where(vv == m, lane, BIGI))
            slot = lo + voff + lsel
            lvc[k // L] = jnp.where(lane == (k % L), m, lvc[k // L])
            lsc[k // L] = jnp.where(lane == (k % L), slot, lsc[k // L])
            vv = jnp.where(lane == lsel, NEG, vv)
            np_v[pl.ds(voff, L)] = vv
            newmax = jnp.max(vv)
            toff = (vsel // L) * L
            lchunk = l1_v[pl.ds(toff, L)]
            l1_v[pl.ds(toff, L)] = jnp.where(
                lane == (vsel - toff), newmax, lchunk)

        cvals_v[pl.ds(0, L)] = lvc[0]
        cvals_v[pl.ds(L, L)] = lvc[1]
        cslots_v[pl.ds(0, L)] = lsc[0]
        cslots_v[pl.ds(L, L)] = lsc[1]
        pltpu.sync_copy(cvals_v.at[pl.ds(0, K)], sh_cvals.at[pl.ds(wid * K, K)])
        pltpu.sync_copy(cslots_v.at[pl.ds(0, K)], sh_cslots.at[pl.ds(wid * K, K)])

    plsc.subcore_barrier()

    @pl.when((cid == 0) & (wid == 0))
    def _phase5():
        pltpu.sync_copy(sh_cvals, cvals_v)
        pltpu.sync_copy(sh_cslots, cslots_v)
        pltpu.sync_copy(sh_win, winall_v)
        ptr = jnp.zeros((L,), jnp.int32)
        rsc = [jnp.full((L,), -1, jnp.int32), jnp.full((L,), -1, jnp.int32)]
        for k in range(K):
            hidx = jnp.minimum(lane * K + ptr, NW * K - 1)
            live = ptr < K
            hv = jnp.where(live, plsc.load_gather(cvals_v, [hidx]), NEG)
            m = jnp.max(hv)
            hsel = jnp.min(jnp.where(hv == m, hidx, BIGI))
            sv = plsc.load_gather(cslots_v, [hidx])
            taken = hidx == hsel
            slot = jnp.max(jnp.where(taken, sv, -1))
            rsc[k // L] = jnp.where(lane == (k % L), slot, rsc[k // L])
            ptr = ptr + jnp.where(taken, 1, 0)
        res_v[pl.ds(0, L)] = rsc[0]
        res_v[pl.ds(L, L)] = rsc[1]
        res_v[pl.ds(2 * L, L)] = plsc.load_gather(winall_v, [rsc[0]])
        res_v[pl.ds(3 * L, L)] = plsc.load_gather(winall_v, [rsc[1]])
        pltpu.sync_copy(res_v, sh_res)

    plsc.subcore_barrier()

    @pl.when(cid == 0)
    def _phase6():
        pltpu.sync_copy(sh_res, res_v)
        bufs = (rowbuf_v, rowbuf2_v)
        for half in range(2):
            schunk = res_v[pl.ds(half * L, L)]
            wchunk = res_v[pl.ds((2 + half) * L, L)]
            slot = jnp.max(jnp.where(lane == wid, schunk, -1))
            win = jnp.max(jnp.where(lane == wid, wchunk, -1))

            @pl.when(win >= 0)
            def _():
                pltpu.make_async_copy(hs_hbm.at[win], bufs[half], sem_row).start()

            @pl.when(win < 0)
            def _():
                pltpu.make_async_copy(mem_hbm.at[slot], bufs[half], sem_row).start()

        for half in range(2):
            pltpu.make_async_copy(hs_hbm.at[0], bufs[half], sem_row).wait()
        for half in range(2):
            row = wid + half * L
            pltpu.make_async_copy(bufs[half], out_hbm.at[row], sem_row).start()
        for half in range(2):
            pltpu.make_async_copy(rowbuf_v, out_hbm.at[0], sem_row).wait()


def kernel(hidden_states, loss, surprise, mem, priorities, idx, targets):
    del targets
    return _sc_replay(idx.astype(jnp.int32), loss, surprise, priorities,
                      hidden_states, mem)


# speedup vs baseline: 1.1054x; 1.0125x over previous
import functools

import jax
import jax.numpy as jnp
from jax import lax
from jax.experimental import pallas as pl
from jax.experimental.pallas import tpu as pltpu
from jax.experimental.pallas import tpu_sc as plsc

CAP = 10000
PAD = 10240
NW = 16
SLOTS_W = PAD // NW
NV = SLOTS_W // 16
B = 1024
L = 16
K = 32
SEQ = 32
HID = 64
NEG = -3.0e38
BIGI = 1 << 30

_mesh = plsc.VectorSubcoreMesh(
    core_axis_name="c", subcore_axis_name="s", num_cores=1, num_subcores=16
)


@functools.partial(
    pl.kernel,
    out_type=jax.ShapeDtypeStruct((K, SEQ, HID), jnp.float32),
    mesh=_mesh,
    scratch_types=[
        pltpu.VMEM((B,), jnp.int32),
        pltpu.VMEM((B,), jnp.float32),
        pltpu.VMEM((B,), jnp.float32),
        pltpu.VMEM((SLOTS_W,), jnp.float32),
        pltpu.VMEM((SLOTS_W,), jnp.int32),
        pltpu.VMEM((2 * L,), jnp.int32),
        pltpu.VMEM((3 * L,), jnp.float32),
        pltpu.VMEM((NW * K,), jnp.float32),
        pltpu.VMEM((NW * K,), jnp.int32),
        pltpu.VMEM((2 * L,), jnp.int32),
        pltpu.VMEM((SEQ, HID), jnp.float32),
        pltpu.VMEM((SEQ, HID), jnp.float32),
        pltpu.VMEM_SHARED((NW * K,), jnp.float32),
        pltpu.VMEM_SHARED((NW * K,), jnp.int32),
        pltpu.VMEM_SHARED((PAD,), jnp.int32),
        pltpu.SemaphoreType.DMA,
        pltpu.SemaphoreType.DMA,
        pltpu.SemaphoreType.DMA,
    ],
    compiler_params=pltpu.CompilerParams(needs_layout_passes=False),
)
def _sc_replay(idx_hbm, loss_hbm, sur_hbm, pri_hbm, hs_hbm, mem_hbm, out_hbm,
               idx_v, loss_v, sur_v, np_v, win_v, shift_v, l1_v,
               cvals_v, cslots_v, tmpw_v, rowbuf_v, rowbuf2_v,
               sh_cvals, sh_cslots, sh_win,
               sem_in, sem_pri, sem_row):
    cid = lax.axis_index("c")
    wid = lax.axis_index("s")
    lo = wid * SLOTS_W
    lane = lax.iota(jnp.int32, L)

    TAIL = CAP - (NW - 1) * SLOTS_W

    @pl.when(cid == 0)
    def _phase1():
        pltpu.make_async_copy(idx_hbm, idx_v, sem_in).start()
        pltpu.make_async_copy(loss_hbm, loss_v, sem_in).start()
        pltpu.make_async_copy(sur_hbm, sur_v, sem_in).start()

        @pl.when(wid == NW - 1)
        def _():
            zf = jnp.zeros((L,), jnp.float32)
            for v in range(TAIL // L, NV):
                np_v[pl.ds(v * L, L)] = zf
            pltpu.make_async_copy(
                pri_hbm.at[pl.ds(lo, TAIL)], np_v.at[pl.ds(0, TAIL)],
                sem_pri).start()

        @pl.when(wid < NW - 1)
        def _():
            pltpu.make_async_copy(
                pri_hbm.at[pl.ds(lo, SLOTS_W)], np_v, sem_pri).start()

        neg1 = jnp.full((L,), -1, jnp.int32)
        for v in range(NV):
            win_v[pl.ds(v * L, L)] = neg1
        shift_v[pl.ds(L, L)] = neg1

        pltpu.make_async_copy(idx_hbm, idx_v, sem_in).wait()
        pltpu.make_async_copy(loss_hbm, loss_v, sem_in).wait()
        pltpu.make_async_copy(sur_hbm, sur_v, sem_in).wait()

        @pl.when(wid == NW - 1)
        def _():
            pltpu.make_async_copy(
                pri_hbm.at[pl.ds(0, TAIL)], np_v.at[pl.ds(0, TAIL)],
                sem_pri).wait()

        @pl.when(wid < NW - 1)
        def _():
            pltpu.make_async_copy(
                pri_hbm.at[pl.ds(0, SLOTS_W)], np_v, sem_pri).wait()

        for c in range(B // L):
            iv = idx_v[pl.ds(c * L, L)]
            pv = (1.0 + loss_v[pl.ds(c * L, L)]) + sur_v[pl.ds(c * L, L)]
            key = iv * L + lane
            sk, spv = plsc.sort_key_val(key, pv)
            sidx = sk >> 4
            sj = (sk & (L - 1)) + (c * L)
            shift_v[pl.ds(0, L)] = sidx
            nxt = shift_v[pl.ds(1, L)]
            keep = sidx != nxt
            rel = sidx - lo
            m = keep & (rel >= 0) & (rel < SLOTS_W)
            relc = jnp.clip(rel, 0, SLOTS_W - 1)
            plsc.store_scatter(np_v, [relc], spv, mask=m)
            plsc.store_scatter(win_v, [relc], sj, mask=m)

        pltpu.make_async_copy(win_v, sh_win.at[pl.ds(lo, SLOTS_W)], sem_pri).start()

        l1c = [jnp.full((L,), NEG) for _ in range(3)]
        for v in range(NV):
            mv = jnp.max(np_v[pl.ds(v * L, L)])
            l1c[v // L] = jnp.where(lane == (v % L), mv, l1c[v // L])
        for t in range(3):
            l1_v[pl.ds(t * L, L)] = l1c[t]

        lvc = [jnp.full((L,), NEG), jnp.full((L,), NEG)]
        lsc = [jnp.full((L,), -1, jnp.int32), jnp.full((L,), -1, jnp.int32)]
        for k in range(K):
            a0 = l1_v[pl.ds(0, L)]
            a1 = l1_v[pl.ds(L, L)]
            a2 = l1_v[pl.ds(2 * L, L)]
            m = jnp.max(jnp.maximum(jnp.maximum(a0, a1), a2))
            vsel = jnp.min(jnp.minimum(
                jnp.minimum(
                    jnp.where(a0 == m, lane, BIGI),
                    jnp.where(a1 == m, lane + L, BIGI)),
                jnp.where(a2 == m, lane + 2 * L, BIGI)))
            voff = vsel * L
            vv = np_v[pl.ds(voff, L)]
            lsel = jnp.min(jnp.where(vv == m, lane, BIGI))
            slot = lo + voff + lsel
            lvc[k // L] = jnp.where(lane == (k % L), m, lvc[k // L])
            lsc[k // L] = jnp.where(lane == (k % L), slot, lsc[k // L])
            vv = jnp.where(lane == lsel, NEG, vv)
            np_v[pl.ds(voff, L)] = vv
            newmax = jnp.max(vv)
            toff = (vsel // L) * L
            lchunk = l1_v[pl.ds(toff, L)]
            l1_v[pl.ds(toff, L)] = jnp.where(
                lane == (vsel - toff), newmax, lchunk)

        cvals_v[pl.ds(0, L)] = lvc[0]
        cvals_v[pl.ds(L, L)] = lvc[1]
        cslots_v[pl.ds(0, L)] = lsc[0]
        cslots_v[pl.ds(L, L)] = lsc[1]
        pltpu.sync_copy(cvals_v.at[pl.ds(0, K)], sh_cvals.at[pl.ds(wid * K, K)])
        pltpu.sync_copy(cslots_v.at[pl.ds(0, K)], sh_cslots.at[pl.ds(wid * K, K)])
        pltpu.make_async_copy(win_v, sh_win.at[pl.ds(lo, SLOTS_W)], sem_pri).wait()

    plsc.subcore_barrier()

    @pl.when(cid == 0)
    def _phase56():
        pltpu.sync_copy(sh_cvals, cvals_v)
        pltpu.sync_copy(sh_cslots, cslots_v)
        ptr = jnp.zeros((L,), jnp.int32)
        rsc = [jnp.full((L,), -1, jnp.int32), jnp.full((L,), -1, jnp.int32)]
        for k in range(K):
            hidx = jnp.minimum(lane * K + ptr, NW * K - 1)
            live = ptr < K
            hv = jnp.where(live, plsc.load_gather(cvals_v, [hidx]), NEG)
            m = jnp.max(hv)
            hsel = jnp.min(jnp.where(hv == m, hidx, BIGI))
            sv = plsc.load_gather(cslots_v, [hidx])
            taken = hidx == hsel
            slot = jnp.max(jnp.where(taken, sv, -1))
            rsc[k // L] = jnp.where(lane == (k % L), slot, rsc[k // L])
            ptr = ptr + jnp.where(taken, 1, 0)

        slots = [None, None]
        wins = [None, None]
        for half in range(2):
            slots[half] = jnp.max(jnp.where(lane == wid, rsc[half], -1))
            base = pl.multiple_of((slots[half] >> 4) << 4, L)
            pltpu.sync_copy(sh_win.at[pl.ds(base, L)],
                            tmpw_v.at[pl.ds(half * L, L)])
            wv = tmpw_v[pl.ds(half * L, L)]
            wins[half] = jnp.max(jnp.where(lane == (slots[half] - base), wv, -1))

        bufs = (rowbuf_v, rowbuf2_v)
        for half in range(2):
            slot, win = slots[half], wins[half]

            @pl.when(win >= 0)
            def _():
                pltpu.make_async_copy(hs_hbm.at[win], bufs[half], sem_row).start()

            @pl.when(win < 0)
            def _():
                pltpu.make_async_copy(mem_hbm.at[slot], bufs[half], sem_row).start()

        for half in range(2):
            pltpu.make_async_copy(hs_hbm.at[0], bufs[half], sem_row).wait()
        for half in range(2):
            row = wid + half * L
            pltpu.make_async_copy(bufs[half], out_hbm.at[row], sem_row).start()
        for half in range(2):
            pltpu.make_async_copy(rowbuf_v, out_hbm.at[0], sem_row).wait()


def kernel(hidden_states, loss, surprise, mem, priorities, idx, targets):
    del targets
    return _sc_replay(idx.astype(jnp.int32), loss, surprise, priorities,
                      hidden_states, mem)


# speedup vs baseline: 1.1077x; 1.0022x over previous
import functools

import jax
import jax.numpy as jnp
from jax import lax
from jax.experimental import pallas as pl
from jax.experimental.pallas import tpu as pltpu
from jax.experimental.pallas import tpu_sc as plsc

CAP = 10000
PAD = 10240
NW = 16
SLOTS_W = PAD // NW
NV = SLOTS_W // 16
B = 1024
L = 16
K = 32
SEQ = 32
HID = 64
NEG = -3.0e38
BIGI = 1 << 30

_mesh = plsc.VectorSubcoreMesh(
    core_axis_name="c", subcore_axis_name="s", num_cores=1, num_subcores=16
)


@functools.partial(
    pl.kernel,
    out_type=jax.ShapeDtypeStruct((K, SEQ, HID), jnp.float32),
    mesh=_mesh,
    scratch_types=[
        pltpu.VMEM((B,), jnp.int32),
        pltpu.VMEM((B,), jnp.float32),
        pltpu.VMEM((B,), jnp.float32),
        pltpu.VMEM((SLOTS_W,), jnp.float32),
        pltpu.VMEM((SLOTS_W,), jnp.int32),
        pltpu.VMEM((2 * L,), jnp.int32),
        pltpu.VMEM((3 * L,), jnp.float32),
        pltpu.VMEM((NW * K,), jnp.float32),
        pltpu.VMEM((NW * K,), jnp.int32),
        pltpu.VMEM((2 * L,), jnp.int32),
        pltpu.VMEM((SEQ, HID), jnp.float32),
        pltpu.VMEM((SEQ, HID), jnp.float32),
        pltpu.VMEM_SHARED((NW * K,), jnp.float32),
        pltpu.VMEM_SHARED((NW * K,), jnp.int32),
        pltpu.VMEM_SHARED((PAD,), jnp.int32),
        pltpu.SemaphoreType.DMA,
        pltpu.SemaphoreType.DMA,
        pltpu.SemaphoreType.DMA,
    ],
    compiler_params=pltpu.CompilerParams(needs_layout_passes=False),
)
def _sc_replay(idx_hbm, loss_hbm, sur_hbm, pri_hbm, hs_hbm, mem_hbm, out_hbm,
               idx_v, loss_v, sur_v, np_v, win_v, shift_v, l1_v,
               cvals_v, cslots_v, tmpw_v, rowbuf_v, rowbuf2_v,
               sh_cvals, sh_cslots, sh_win,
               sem_in, sem_pri, sem_row):
    cid = lax.axis_index("c")
    wid = lax.axis_index("s")
    lo = wid * SLOTS_W
    lane = lax.iota(jnp.int32, L)

    TAIL = CAP - (NW - 1) * SLOTS_W

    @pl.when(cid == 0)
    def _phase1():
        pltpu.make_async_copy(idx_hbm, idx_v, sem_in).start()
        pltpu.make_async_copy(loss_hbm, loss_v, sem_in).start()
        pltpu.make_async_copy(sur_hbm, sur_v, sem_in).start()

        @pl.when(wid == NW - 1)
        def _():
            zf = jnp.zeros((L,), jnp.float32)
            for v in range(TAIL // L, NV):
                np_v[pl.ds(v * L, L)] = zf
            pltpu.make_async_copy(
                pri_hbm.at[pl.ds(lo, TAIL)], np_v.at[pl.ds(0, TAIL)],
                sem_pri).start()

        @pl.when(wid < NW - 1)
        def _():
            pltpu.make_async_copy(
                pri_hbm.at[pl.ds(lo, SLOTS_W)], np_v, sem_pri).start()

        neg1 = jnp.full((L,), -1, jnp.int32)
        for v in range(NV):
            win_v[pl.ds(v * L, L)] = neg1
        shift_v[pl.ds(L, L)] = neg1

        pltpu.make_async_copy(idx_hbm, idx_v, sem_in).wait()
        pltpu.make_async_copy(loss_hbm, loss_v, sem_in).wait()
        pltpu.make_async_copy(sur_hbm, sur_v, sem_in).wait()

        @pl.when(wid == NW - 1)
        def _():
            pltpu.make_async_copy(
                pri_hbm.at[pl.ds(0, TAIL)], np_v.at[pl.ds(0, TAIL)],
                sem_pri).wait()

        @pl.when(wid < NW - 1)
        def _():
            pltpu.make_async_copy(
                pri_hbm.at[pl.ds(0, SLOTS_W)], np_v, sem_pri).wait()

        for c in range(B // L):
            iv = idx_v[pl.ds(c * L, L)]
            pv = (1.0 + loss_v[pl.ds(c * L, L)]) + sur_v[pl.ds(c * L, L)]
            key = iv * L + lane
            sk, spv = plsc.sort_key_val(key, pv)
            sidx = sk >> 4
            sj = (sk & (L - 1)) + (c * L)
            shift_v[pl.ds(0, L)] = sidx
            nxt = shift_v[pl.ds(1, L)]
            keep = sidx != nxt
            rel = sidx - lo
            m = keep & (rel >= 0) & (rel < SLOTS_W)
            relc = jnp.clip(rel, 0, SLOTS_W - 1)
            plsc.store_scatter(np_v, [relc], spv, mask=m)
            plsc.store_scatter(win_v, [relc], sj, mask=m)

        pltpu.make_async_copy(win_v, sh_win.at[pl.ds(lo, SLOTS_W)], sem_pri).start()

        l1c = [jnp.full((L,), NEG) for _ in range(3)]
        for v in range(NV):
            mv = jnp.max(np_v[pl.ds(v * L, L)])
            l1c[v // L] = jnp.where(lane == (v % L), mv, l1c[v // L])
        for t in range(3):
            l1_v[pl.ds(t * L, L)] = l1c[t]

        lvc = [jnp.full((L,), NEG), jnp.full((L,), NEG)]
        lsc = [jnp.full((L,), -1, jnp.int32), jnp.full((L,), -1, jnp.int32)]
        for k in range(K):
            a0 = l1_v[pl.ds(0, L)]
            a1 = l1_v[pl.ds(L, L)]
            a2 = l1_v[pl.ds(2 * L, L)]
            m = jnp.max(jnp.maximum(jnp.maximum(a0, a1), a2))
            vsel = jnp.min(jnp.minimum(
                jnp.minimum(
                    jnp.where(a0 == m, lane, BIGI),
                    jnp.where(a1 == m, lane + L, BIGI)),
                jnp.where(a2 == m, lane + 2 * L, BIGI)))
            voff = vsel * L
            vv = np_v[pl.ds(voff, L)]
            lsel = jnp.min(jnp.where(vv == m, lane, BIGI))
            slot = lo + voff + lsel
            lvc[k // L] = jnp.where(lane == (k % L), m, lvc[k // L])
            lsc[k // L] = jnp.where(lane == (k % L), slot, lsc[k // L])
            vv = jnp.where(lane == lsel, NEG, vv)
            np_v[pl.ds(voff, L)] = vv
            newmax = jnp.max(vv)
            toff = (vsel // L) * L
            lchunk = l1_v[pl.ds(toff, L)]
            l1_v[pl.ds(toff, L)] = jnp.where(
                lane == (vsel - toff), newmax, lchunk)

        cvals_v[pl.ds(0, L)] = lvc[0]
        cvals_v[pl.ds(L, L)] = lvc[1]
        cslots_v[pl.ds(0, L)] = lsc[0]
        cslots_v[pl.ds(L, L)] = lsc[1]
        pltpu.make_async_copy(
            cvals_v.at[pl.ds(0, K)], sh_cvals.at[pl.ds(wid * K, K)],
            sem_in).start()
        pltpu.make_async_copy(
            cslots_v.at[pl.ds(0, K)], sh_cslots.at[pl.ds(wid * K, K)],
            sem_in).start()
        pltpu.make_async_copy(
            cvals_v.at[pl.ds(0, K)], sh_cvals.at[pl.ds(0, K)], sem_in).wait()
        pltpu.make_async_copy(
            cslots_v.at[pl.ds(0, K)], sh_cslots.at[pl.ds(0, K)], sem_in).wait()
        pltpu.make_async_copy(win_v, sh_win.at[pl.ds(lo, SLOTS_W)], sem_pri).wait()

    plsc.subcore_barrier()

    @pl.when(cid == 0)
    def _phase56():
        pltpu.make_async_copy(sh_cvals, cvals_v, sem_in).start()
        pltpu.make_async_copy(sh_cslots, cslots_v, sem_in).start()
        pltpu.make_async_copy(sh_cvals, cvals_v, sem_in).wait()
        pltpu.make_async_copy(sh_cslots, cslots_v, sem_in).wait()
        ptr = jnp.zeros((L,), jnp.int32)
        rsc = [jnp.full((L,), -1, jnp.int32), jnp.full((L,), -1, jnp.int32)]
        for k in range(K):
            hidx = jnp.minimum(lane * K + ptr, NW * K - 1)
            live = ptr < K
            hv = jnp.where(live, plsc.load_gather(cvals_v, [hidx]), NEG)
            m = jnp.max(hv)
            hsel = jnp.min(jnp.where(hv == m, hidx, BIGI))
            sv = plsc.load_gather(cslots_v, [hidx])
            taken = hidx == hsel
            slot = jnp.max(jnp.where(taken, sv, -1))
            rsc[k // L] = jnp.where(lane == (k % L), slot, rsc[k // L])
            ptr = ptr + jnp.where(taken, 1, 0)

        slots = [None, None]
        wins = [None, None]
        for half in range(2):
            slots[half] = jnp.max(jnp.where(lane == wid, rsc[half], -1))
            base = pl.multiple_of((slots[half] >> 4) << 4, L)
            pltpu.sync_copy(sh_win.at[pl.ds(base, L)],
                            tmpw_v.at[pl.ds(half * L, L)])
            wv = tmpw_v[pl.ds(half * L, L)]
            wins[half] = jnp.max(jnp.where(lane == (slots[half] - base), wv, -1))

        bufs = (rowbuf_v, rowbuf2_v)
        for half in range(2):
            slot, win = slots[half], wins[half]

            @pl.when(win >= 0)
            def _():
                pltpu.make_async_copy(hs_hbm.at[win], bufs[half], sem_row).start()

            @pl.when(win < 0)
            def _():
                pltpu.make_async_copy(mem_hbm.at[slot], bufs[half], sem_row).start()

        for half in range(2):
            pltpu.make_async_copy(hs_hbm.at[0], bufs[half], sem_row).wait()
        for half in range(2):
            row = wid + half * L
            pltpu.make_async_copy(bufs[half], out_hbm.at[row], sem_row).start()
        for half in range(2):
            pltpu.make_async_copy(rowbuf_v, out_hbm.at[0], sem_row).wait()


def kernel(hidden_states, loss, surprise, mem, priorities, idx, targets):
    del targets
    return _sc_replay(idx.astype(jnp.int32), loss, surprise, priorities,
                      hidden_states, mem)
